# baseline (speedup 1.0000x reference)
import functools

import jax
import jax.numpy as jnp
from jax import lax
from jax.experimental import pallas as pl
from jax.experimental.pallas import tpu as pltpu
from jax.experimental.pallas import tpu_sc as plsc

_CH = 32768
_QPP = 4
_NPLANES = 104
_NW = 32
_QPW = _NPLANES * _QPP // _NW
_L = 16


def _sc_body(ptab_hbm, z_hbm, o_hbm, ptab_v, buf, sems):
    wid = lax.axis_index("s") * 2 + lax.axis_index("c")
    pltpu.sync_copy(ptab_hbm, ptab_v)

    def compute_chunk(b, q):
        base = (q // _QPP) * 4 * _L
        C2 = ptab_v[pl.ds(base, _L)]
        D2 = ptab_v[pl.ds(base + _L, _L)]
        P = ptab_v[pl.ds(base + 2 * _L, _L)]
        Q = ptab_v[pl.ds(base + 3 * _L, _L)]

        def step(i, carry):
            x = buf[b, pl.ds(i * _L, _L)]
            u = jnp.exp(x * C2 - D2)
            buf[b, pl.ds(i * _L, _L)] = P - Q / (u + 1.0)
            return carry

        lax.fori_loop(0, _CH // _L, step, 0)

    q0 = wid * _QPW
    pltpu.async_copy(
        z_hbm.at[pl.ds(q0 * _CH, _CH)], buf.at[0], sems.at[0]).wait()
    for j in range(_QPW):
        b = j % 2
        q = q0 + j
        if j + 1 < _QPW:
            nxt = pltpu.async_copy(
                z_hbm.at[pl.ds((q + 1) * _CH, _CH)], buf.at[1 - b],
                sems.at[1 - b])
        compute_chunk(b, q)
        pltpu.async_copy(
            buf.at[b], o_hbm.at[pl.ds(q * _CH, _CH)], sems.at[b]).wait()
        if j + 1 < _QPW:
            nxt.wait()


@functools.partial(jax.jit, static_argnames=("interpret",))
def kernel(z, Mask, eta_table, interpret=False):
    del interpret
    M, N, S, F = z.shape
    E = M * F * N * S
    zt = jnp.transpose(z, (0, 3, 1, 2)).reshape(E)
    eta = jnp.take(eta_table, Mask.reshape(M * F), axis=0)
    e0, e1, e2, e3 = (eta[:, k] for k in range(4))
    params = jnp.stack([2.0 * e3, 2.0 * e2 * e3, -e0 - e1, -2.0 * e1], axis=1)
    ptab = jnp.broadcast_to(params[:, :, None],
                            (_NPLANES, 4, _L)).reshape(_NPLANES * 4 * _L)
    mesh = plsc.VectorSubcoreMesh(core_axis_name="c", subcore_axis_name="s")
    out = pl.kernel(
        _sc_body,
        mesh=mesh,
        out_type=jax.ShapeDtypeStruct((E,), jnp.float32),
        scratch_types=[
            pltpu.VMEM((_NPLANES * 4 * _L,), jnp.float32),
            pltpu.VMEM((2, _CH), jnp.float32),
            pltpu.SemaphoreType.DMA((2,)),
        ],
    )(ptab, zt)
    return out.reshape(M, F, N, S).transpose(0, 2, 3, 1)


# speedup vs baseline: 17.2830x; 17.2830x over previous
import functools

import jax
import jax.numpy as jnp
from jax.experimental import pallas as pl
from jax.experimental.pallas import tpu as pltpu

_NBUF = 13
_RB = 2048
_PPC = 2
_NCHUNKS = 52


def _body(mask_ref, eta_ref, z_ref, o_ref, ibuf, obuf, isem, osem):
    def start_in(c, k):
        pltpu.make_async_copy(
            z_ref.at[pl.ds(c * _RB, _RB), :], ibuf.at[k], isem.at[k]).start()

    def wait_in(c, k):
        pltpu.make_async_copy(
            z_ref.at[pl.ds(c * _RB, _RB), :], ibuf.at[k], isem.at[k]).wait()

    def start_out(c, k):
        pltpu.make_async_copy(
            obuf.at[k], o_ref.at[pl.ds(c * _RB, _RB), :], osem.at[k]).start()

    def wait_out(c, k):
        pltpu.make_async_copy(
            obuf.at[k], o_ref.at[pl.ds(c * _RB, _RB), :], osem.at[k]).wait()

    def compute(c, k):
        for h in range(_PPC):
            t = mask_ref[_PPC * c + h]
            A = -eta_ref[t, 0]
            B = -eta_ref[t, 1]
            C = eta_ref[t, 3]
            D = eta_ref[t, 2] * C
            x = ibuf[k, h * 1024:(h + 1) * 1024, :]
            obuf[k, h * 1024:(h + 1) * 1024, :] = A + B * jnp.tanh(x * C - D)

    for k in range(_NBUF):
        start_in(k, k)

    for k in range(_NBUF):
        wait_in(k, k)
        compute(k, k)
        start_in(k + _NBUF, k)
        start_out(k, k)

    def group(g, carry):
        for k in range(_NBUF):
            c = g * _NBUF + k
            wait_in(c, k)
            wait_out(c - _NBUF, k)
            compute(c, k)
            start_in(c + _NBUF, k)
            start_out(c, k)
        return carry

    n_groups = _NCHUNKS // _NBUF
    jax.lax.fori_loop(1, n_groups - 1, group, 0)

    for k in range(_NBUF):
        c = (n_groups - 1) * _NBUF + k
        wait_in(c, k)
        wait_out(c - _NBUF, k)
        compute(c, k)
        start_out(c, k)

    for k in range(_NBUF):
        wait_out((n_groups - 1) * _NBUF + k, k)


@functools.partial(jax.jit, static_argnames=("interpret",))
def kernel(z, Mask, eta_table, interpret=False):
    M, N, S, F = z.shape
    R = M * F * N
    zt = jnp.transpose(z, (0, 3, 1, 2)).reshape(R, S)
    mask_flat = Mask.astype(jnp.int32).reshape(M * F)
    out = pl.pallas_call(
        _body,
        in_specs=[
            pl.BlockSpec(memory_space=pltpu.SMEM),
            pl.BlockSpec(memory_space=pltpu.SMEM),
            pl.BlockSpec(memory_space=pltpu.MemorySpace.HBM),
        ],
        out_specs=pl.BlockSpec(memory_space=pltpu.MemorySpace.HBM),
        out_shape=jax.ShapeDtypeStruct((R, S), jnp.float32),
        scratch_shapes=[
            pltpu.VMEM((_NBUF, _RB, S), jnp.float32),
            pltpu.VMEM((_NBUF, _RB, S), jnp.float32),
            pltpu.SemaphoreType.DMA((_NBUF,)),
            pltpu.SemaphoreType.DMA((_NBUF,)),
        ],
        interpret=interpret,
    )(mask_flat, eta_table, zt)
    return out.reshape(M, F, N, S).transpose(0, 2, 3, 1)
